# Initial kernel scaffold; baseline (speedup 1.0000x reference)
import jax
import jax.numpy as jnp
from jax.experimental import pallas as pl


def kernel(src_nodes, dst_nodes, neg_nodes, edge_times, node_features, node_timestamps, neighbors, time_w, time_b, sage_w, sage_b, fc_w, fc_b, merge_w1, merge_b1, merge_w2, merge_b2):
    raise NotImplementedError("write your pallas kernel here")


# speedup vs baseline: 4.2916x; 4.2916x over previous
import functools

import jax
import jax.numpy as jnp
from jax import lax
from jax.experimental import pallas as pl
from jax.experimental.pallas import tpu as pltpu
from jax.experimental.pallas import tpu_sc as plsc

N_NODES = 100000
D = 128
B = 4096
K = 20
NB3 = 3 * B
NC = 2
NS = 16
NW = NC * NS
PER_W = NB3 // NW
C = 32
N_CHUNKS = PER_W // C
KP = 32


def _sc_gather_body(feat_hbm, ts_hbm, nbr_hbm, nodes_hbm,
                    self_out, nsum_out, nt_out, tn_out,
                    ts_v, idx_v, nidx_v, idxrow_v, self_v, rows_v, nsum_v,
                    nt_v, tn_v, sem_nbr, sem_self, sem_rows):
    wid = lax.axis_index("s") * NC + lax.axis_index("c")
    base = wid * PER_W

    pltpu.sync_copy(ts_hbm, ts_v)

    @pl.loop(0, N_CHUNKS)
    def _chunk(c):
        cb = base + c * C
        pltpu.sync_copy(nodes_hbm.at[pl.ds(cb, C)], idx_v)
        nbr_cp = pltpu.async_copy(nbr_hbm.at[idx_v], nidx_v, sem_nbr)
        self_cp = pltpu.async_copy(feat_hbm.at[idx_v], self_v, sem_self)
        nbr_cp.wait()
        self_cp.wait()
        pltpu.sync_copy(self_v, self_out.at[pl.ds(cb, C)])

        for h in range(C // 16):
            iv = idx_v[pl.ds(h * 16, 16)]
            tn_v[pl.ds(h * 16, 16)] = plsc.load_gather(ts_v, [iv])

        @pl.loop(0, C)
        def _node(i):
            ia = nidx_v[i, pl.ds(0, 16)]
            ib = nidx_v[i, pl.ds(K - 16, 16)]
            idxrow_v[pl.ds(0, 16)] = ia
            idxrow_v[pl.ds(K - 16, 16)] = ib
            cp = pltpu.async_copy(feat_hbm.at[idxrow_v], rows_v, sem_rows)
            nt_v[i, pl.ds(0, 16)] = plsc.load_gather(ts_v, [ia])
            nt_v[i, pl.ds(K - 16, 16)] = plsc.load_gather(ts_v, [ib])
            cp.wait()
            for l in range(D // 16):
                acc = rows_v[0, pl.ds(l * 16, 16)]
                for k in range(1, K):
                    acc = acc + rows_v[k, pl.ds(l * 16, 16)]
                nsum_v[i, pl.ds(l * 16, 16)] = acc

        pltpu.sync_copy(nsum_v, nsum_out.at[pl.ds(cb, C)])
        pltpu.sync_copy(nt_v, nt_out.at[pl.ds(cb, C)])
        pltpu.sync_copy(tn_v, tn_out.at[pl.ds(cb, C)])


def _sc_gather(feat, ts, nbr, nodes):
    mesh = plsc.VectorSubcoreMesh(core_axis_name="c", subcore_axis_name="s",
                                  num_cores=NC, num_subcores=NS)
    f = pl.kernel(
        _sc_gather_body,
        compiler_params=pltpu.CompilerParams(needs_layout_passes=False,
                                             use_tc_tiling_on_sc=False),
        out_type=(
            jax.ShapeDtypeStruct((NB3, D), jnp.float32),
            jax.ShapeDtypeStruct((NB3, D), jnp.float32),
            jax.ShapeDtypeStruct((NB3, K), jnp.float32),
            jax.ShapeDtypeStruct((NB3,), jnp.float32),
        ),
        mesh=mesh,
        scratch_types=[
            pltpu.VMEM((N_NODES,), jnp.float32),
            pltpu.VMEM((C,), jnp.int32),
            pltpu.VMEM((C, KP), jnp.int32),
            pltpu.VMEM((K,), jnp.int32),
            pltpu.VMEM((C, D), jnp.float32),
            pltpu.VMEM((K, D), jnp.float32),
            pltpu.VMEM((C, D), jnp.float32),
            pltpu.VMEM((C, K), jnp.float32),
            pltpu.VMEM((C,), jnp.float32),
            pltpu.SemaphoreType.DMA,
            pltpu.SemaphoreType.DMA,
            pltpu.SemaphoreType.DMA,
        ],
    )
    return f(feat, ts, nbr, nodes)


RB = 512
GRID = B // RB


def _tc_body(selff_ref, nsum_ref, nt_ref, tn_ref, et_ref,
             tw_ref, tb_ref, sw_ref, sb_ref, fw_ref, fb_ref,
             m1_ref, mb1_ref, w2_ref, mb2_ref,
             pos_ref, neg_ref):
    w = tw_ref[0:1, :]
    bvec = tb_ref[0:1, :]
    et = et_ref[:, :]

    ne = []
    for s in range(3):
        nt = nt_ref[s]
        delta = et - nt
        csum = jnp.zeros((RB, D), jnp.float32)
        for k in range(K):
            csum = csum + jnp.cos(delta[:, k:k + 1] * w + bvec)
        agg = (nsum_ref[s] + csum) * (1.0 / K)
        sf = selff_ref[s]
        ge = jnp.maximum(
            jnp.dot(sf, sw_ref[0:D, :], preferred_element_type=jnp.float32)
            + jnp.dot(agg, sw_ref[D:2 * D, :], preferred_element_type=jnp.float32)
            + sb_ref[0:1, :], 0.0)
        te = sf + jnp.cos(tn_ref[s] * w + bvec)
        ne.append(
            jnp.dot(te, fw_ref[0:D, :], preferred_element_type=jnp.float32)
            + jnp.dot(ge, fw_ref[D:2 * D, :], preferred_element_type=jnp.float32)
            + fb_ref[0:1, :])

    a = jnp.dot(ne[0], m1_ref[0:D, :], preferred_element_type=jnp.float32)
    hp = jnp.maximum(a + jnp.dot(ne[1], m1_ref[D:2 * D, :],
                                 preferred_element_type=jnp.float32)
                     + mb1_ref[0:1, :], 0.0)
    hn = jnp.maximum(a + jnp.dot(ne[2], m1_ref[D:2 * D, :],
                                 preferred_element_type=jnp.float32)
                     + mb1_ref[0:1, :], 0.0)
    w2 = w2_ref[0:1, :]
    b2 = mb2_ref[0, 0]
    sp = jnp.sum(hp * w2, axis=1) + b2
    sn = jnp.sum(hn * w2, axis=1) + b2
    pos_ref[:, :] = jax.nn.sigmoid(sp)[:, None]
    neg_ref[:, :] = jax.nn.sigmoid(sn)[:, None]


def _tc_dense(selff, nsum, nt, tn, et, tw, tb, sw, sb, fw, fb, m1, mb1, w2, mb2):
    full = lambda shape: pl.BlockSpec(shape, lambda j: (0,) * len(shape))
    grid_spec = pl.GridSpec(
        grid=(GRID,),
        in_specs=[
            pl.BlockSpec((3, RB, D), lambda j: (0, j, 0)),
            pl.BlockSpec((3, RB, D), lambda j: (0, j, 0)),
            pl.BlockSpec((3, RB, K), lambda j: (0, j, 0)),
            pl.BlockSpec((3, RB, 1), lambda j: (0, j, 0)),
            pl.BlockSpec((RB, 1), lambda j: (j, 0)),
            full((1, D)), full((1, D)),
            full((2 * D, D)), full((1, D)),
            full((2 * D, D)), full((1, D)),
            full((2 * D, D)), full((1, D)),
            full((1, D)), full((1, 1)),
        ],
        out_specs=[
            pl.BlockSpec((RB, 1), lambda j: (j, 0)),
            pl.BlockSpec((RB, 1), lambda j: (j, 0)),
        ],
    )
    return pl.pallas_call(
        _tc_body,
        grid_spec=grid_spec,
        out_shape=(
            jax.ShapeDtypeStruct((B, 1), jnp.float32),
            jax.ShapeDtypeStruct((B, 1), jnp.float32),
        ),
    )(selff, nsum, nt, tn, et, tw, tb, sw, sb, fw, fb, m1, mb1, w2, mb2)


def kernel(src_nodes, dst_nodes, neg_nodes, edge_times, node_features,
           node_timestamps, neighbors, time_w, time_b, sage_w, sage_b,
           fc_w, fc_b, merge_w1, merge_b1, merge_w2, merge_b2):
    nodes = jnp.concatenate([src_nodes, dst_nodes, neg_nodes]).astype(jnp.int32)
    nbr_pad = jnp.pad(neighbors.astype(jnp.int32), ((0, 0), (0, KP - K)))
    selff, nsum, nt, tn = _sc_gather(node_features, node_timestamps,
                                     nbr_pad, nodes)
    pos, neg = _tc_dense(
        selff.reshape(3, B, D), nsum.reshape(3, B, D), nt.reshape(3, B, K),
        tn.reshape(3, B, 1), edge_times.reshape(B, 1),
        time_w.reshape(1, D), time_b.reshape(1, D),
        sage_w, sage_b.reshape(1, D), fc_w, fc_b.reshape(1, D),
        merge_w1, merge_b1.reshape(1, D), merge_w2.reshape(1, D),
        merge_b2.reshape(1, 1))
    return pos.reshape(B), neg.reshape(B)


# speedup vs baseline: 9.2174x; 2.1478x over previous
import functools

import jax
import jax.numpy as jnp
from jax import lax
from jax.experimental import pallas as pl
from jax.experimental.pallas import tpu as pltpu
from jax.experimental.pallas import tpu_sc as plsc

N_NODES = 100000
D = 128
B = 4096
K = 20
NC = 2
NS = 16
NW = NC * NS
PER_W = B // NW
C = 16
N_CHUNKS = PER_W // C
KP = 32
NPD = 4
NSB = C // NPD


def _sc_gather_body(feat_hbm, ts_hbm, nbr_hbm, nodes_hbm,
                    self_out, nsum_out, nt_out, tn_out,
                    ts_v, idx_v, nidx_v, idxblk_v, self_v, rows_v, nsum_v,
                    nt_v, tn_v, sem_nbr, sem_self, sem_a, sem_b):
    wid = lax.axis_index("s") * NC + lax.axis_index("c")
    base = wid * PER_W
    sems = (sem_a, sem_b)

    pltpu.sync_copy(ts_hbm, ts_v)

    def _stage(sb):
        b = sb % 2
        for j in range(NPD):
            n = sb * NPD + j
            ia = nidx_v[n, pl.ds(0, 16)]
            ib = nidx_v[n, pl.ds(K - 16, 16)]
            idxblk_v[b, pl.ds(j * K, 16)] = ia
            idxblk_v[b, pl.ds(j * K + K - 16, 16)] = ib
            nt_v[n, pl.ds(0, 16)] = plsc.load_gather(ts_v, [ia])
            nt_v[n, pl.ds(K - 16, 16)] = plsc.load_gather(ts_v, [ib])
        return pltpu.async_copy(feat_hbm.at[idxblk_v.at[b]], rows_v.at[b],
                                sems[b])

    def _consume(sb):
        b = sb % 2
        pltpu.make_async_copy(feat_hbm.at[idxblk_v.at[b]], rows_v.at[b],
                              sems[b]).wait()

        @pl.loop(0, NPD)
        def _node(j):
            n = sb * NPD + j
            r0 = j * K
            for l in range(D // 16):
                acc = rows_v[b, r0, pl.ds(l * 16, 16)]
                for k in range(1, K):
                    acc = acc + rows_v[b, r0 + k, pl.ds(l * 16, 16)]
                nsum_v[n, pl.ds(l * 16, 16)] = acc

    @pl.loop(0, N_CHUNKS)
    def _chunk(c):
        cb = base + c * C
        pltpu.sync_copy(nodes_hbm.at[pl.ds(cb, C)], idx_v)
        nbr_cp = pltpu.async_copy(nbr_hbm.at[idx_v], nidx_v, sem_nbr)
        self_cp = pltpu.async_copy(feat_hbm.at[idx_v], self_v, sem_self)
        nbr_cp.wait()

        iv = idx_v[pl.ds(0, 16)]
        tn_v[pl.ds(0, 16)] = plsc.load_gather(ts_v, [iv])

        _stage(0)
        for sb in range(NSB):
            if sb + 1 < NSB:
                _stage(sb + 1)
            _consume(sb)

        self_cp.wait()
        pltpu.sync_copy(self_v, self_out.at[pl.ds(cb, C)])
        pltpu.sync_copy(nsum_v, nsum_out.at[pl.ds(cb, C)])
        pltpu.sync_copy(nt_v, nt_out.at[pl.ds(cb, C)])
        pltpu.sync_copy(tn_v, tn_out.at[pl.ds(cb, C)])


def _sc_gather(feat, ts, nbr, nodes):
    mesh = plsc.VectorSubcoreMesh(core_axis_name="c", subcore_axis_name="s",
                                  num_cores=NC, num_subcores=NS)
    f = pl.kernel(
        _sc_gather_body,
        compiler_params=pltpu.CompilerParams(needs_layout_passes=False,
                                             use_tc_tiling_on_sc=False),
        out_type=(
            jax.ShapeDtypeStruct((B, D), jnp.float32),
            jax.ShapeDtypeStruct((B, D), jnp.float32),
            jax.ShapeDtypeStruct((B, K), jnp.float32),
            jax.ShapeDtypeStruct((B,), jnp.float32),
        ),
        mesh=mesh,
        scratch_types=[
            pltpu.VMEM((N_NODES,), jnp.float32),
            pltpu.VMEM((C,), jnp.int32),
            pltpu.VMEM((C, KP), jnp.int32),
            pltpu.VMEM((2, NPD * K), jnp.int32),
            pltpu.VMEM((C, D), jnp.float32),
            pltpu.VMEM((2, NPD * K, D), jnp.float32),
            pltpu.VMEM((C, D), jnp.float32),
            pltpu.VMEM((C, K), jnp.float32),
            pltpu.VMEM((C,), jnp.float32),
            pltpu.SemaphoreType.DMA,
            pltpu.SemaphoreType.DMA,
            pltpu.SemaphoreType.DMA,
            pltpu.SemaphoreType.DMA,
        ],
    )
    return f(feat, ts, nbr, nodes)


RB = 512
GRID = B // RB

_INV2PI = 0.15915494309189535
_C2PI_HI = 6.28125
_C2PI_LO = 0.0019353071795864769
_COS_C = (0.9999999922836861, -0.49999991772810104, 0.04166652436596424,
          -0.001388797041345639, 2.477342421737267e-05,
          -2.7113373320120093e-07, 1.7369133908928802e-09)


def _fast_cos(x):
    k = jnp.rint(x * _INV2PI)
    r = (x - k * _C2PI_HI) - k * _C2PI_LO
    r2 = r * r
    acc = jnp.float32(_COS_C[6])
    for j in range(5, -1, -1):
        acc = acc * r2 + jnp.float32(_COS_C[j])
    return acc


def _tc_embed_body(selff_ref, nsum_ref, nt_ref, tn_ref, et_ref,
                   tw_ref, tb_ref, sw_ref, sb_ref, fw_ref, fb_ref, ne_ref):
    w = tw_ref[0:1, :]
    bvec = tb_ref[0:1, :]
    et = et_ref[:, :]

    nt = nt_ref[:, :]
    delta = et - nt
    csum = jnp.zeros((RB, D), jnp.float32)
    for k in range(K):
        csum = csum + _fast_cos(delta[:, k:k + 1] * w + bvec)
    agg = (nsum_ref[:, :] + csum) * (1.0 / K)
    sf = selff_ref[:, :]
    ge = jnp.maximum(
        jnp.dot(sf, sw_ref[0:D, :], preferred_element_type=jnp.float32)
        + jnp.dot(agg, sw_ref[D:2 * D, :], preferred_element_type=jnp.float32)
        + sb_ref[0:1, :], 0.0)
    te = sf + _fast_cos(tn_ref[:, :] * w + bvec)
    ne_ref[:, :] = (
        jnp.dot(te, fw_ref[0:D, :], preferred_element_type=jnp.float32)
        + jnp.dot(ge, fw_ref[D:2 * D, :], preferred_element_type=jnp.float32)
        + fb_ref[0:1, :])


def _tc_embed(selff, nsum, nt, tn, et, tw, tb, sw, sb, fw, fb):
    full = lambda shape: pl.BlockSpec(shape, lambda j: (0,) * len(shape))
    grid_spec = pl.GridSpec(
        grid=(GRID,),
        in_specs=[
            pl.BlockSpec((RB, D), lambda j: (j, 0)),
            pl.BlockSpec((RB, D), lambda j: (j, 0)),
            pl.BlockSpec((RB, K), lambda j: (j, 0)),
            pl.BlockSpec((RB, 1), lambda j: (j, 0)),
            pl.BlockSpec((RB, 1), lambda j: (j, 0)),
            full((1, D)), full((1, D)),
            full((2 * D, D)), full((1, D)),
            full((2 * D, D)), full((1, D)),
        ],
        out_specs=pl.BlockSpec((RB, D), lambda j: (j, 0)),
    )
    return pl.pallas_call(
        _tc_embed_body,
        grid_spec=grid_spec,
        out_shape=jax.ShapeDtypeStruct((B, D), jnp.float32),
    )(selff, nsum, nt, tn, et, tw, tb, sw, sb, fw, fb)


def _tc_merge_body(ne0_ref, ne1_ref, ne2_ref, m1_ref, mb1_ref, w2_ref,
                   mb2_ref, pos_ref, neg_ref):
    a = jnp.dot(ne0_ref[:, :], m1_ref[0:D, :],
                preferred_element_type=jnp.float32)
    hp = jnp.maximum(a + jnp.dot(ne1_ref[:, :], m1_ref[D:2 * D, :],
                                 preferred_element_type=jnp.float32)
                     + mb1_ref[0:1, :], 0.0)
    hn = jnp.maximum(a + jnp.dot(ne2_ref[:, :], m1_ref[D:2 * D, :],
                                 preferred_element_type=jnp.float32)
                     + mb1_ref[0:1, :], 0.0)
    w2 = w2_ref[0:1, :]
    b2 = mb2_ref[0, 0]
    sp = jnp.sum(hp * w2, axis=1) + b2
    sn = jnp.sum(hn * w2, axis=1) + b2
    pos_ref[:, :] = jax.nn.sigmoid(sp)[:, None]
    neg_ref[:, :] = jax.nn.sigmoid(sn)[:, None]


def _tc_merge(ne0, ne1, ne2, m1, mb1, w2, mb2):
    full = lambda shape: pl.BlockSpec(shape, lambda j: (0,) * len(shape))
    grid_spec = pl.GridSpec(
        grid=(GRID,),
        in_specs=[
            pl.BlockSpec((RB, D), lambda j: (j, 0)),
            pl.BlockSpec((RB, D), lambda j: (j, 0)),
            pl.BlockSpec((RB, D), lambda j: (j, 0)),
            full((2 * D, D)), full((1, D)),
            full((1, D)), full((1, 1)),
        ],
        out_specs=[
            pl.BlockSpec((RB, 1), lambda j: (j, 0)),
            pl.BlockSpec((RB, 1), lambda j: (j, 0)),
        ],
    )
    return pl.pallas_call(
        _tc_merge_body,
        grid_spec=grid_spec,
        out_shape=(
            jax.ShapeDtypeStruct((B, 1), jnp.float32),
            jax.ShapeDtypeStruct((B, 1), jnp.float32),
        ),
    )(ne0, ne1, ne2, m1, mb1, w2, mb2)


def kernel(src_nodes, dst_nodes, neg_nodes, edge_times, node_features,
           node_timestamps, neighbors, time_w, time_b, sage_w, sage_b,
           fc_w, fc_b, merge_w1, merge_b1, merge_w2, merge_b2):
    nbr_pad = jnp.pad(neighbors.astype(jnp.int32), ((0, 0), (0, KP - K)))
    et = edge_times.reshape(B, 1)
    tw = time_w.reshape(1, D)
    tb = time_b.reshape(1, D)
    sb = sage_b.reshape(1, D)
    fb = fc_b.reshape(1, D)

    ne = []
    for nodes in (src_nodes, dst_nodes, neg_nodes):
        selff, nsum, nt, tn = _sc_gather(node_features, node_timestamps,
                                         nbr_pad, nodes.astype(jnp.int32))
        ne.append(_tc_embed(selff, nsum, nt, tn.reshape(B, 1), et,
                            tw, tb, sage_w, sb, fc_w, fb))

    pos, neg = _tc_merge(ne[0], ne[1], ne[2], merge_w1,
                         merge_b1.reshape(1, D), merge_w2.reshape(1, D),
                         merge_b2.reshape(1, 1))
    return pos.reshape(B), neg.reshape(B)


# speedup vs baseline: 9.2204x; 1.0003x over previous
import functools

import jax
import jax.numpy as jnp
from jax import lax
from jax.experimental import pallas as pl
from jax.experimental.pallas import tpu as pltpu
from jax.experimental.pallas import tpu_sc as plsc

N_NODES = 100000
D = 128
B = 4096
K = 20
NC = 2
NS = 16
NW = NC * NS
PER_W = B // NW
C = 16
N_CHUNKS = PER_W // C
KP = 32
NPD = 4
NSB = C // NPD


def _sc_gather_body(feat_hbm, ts_hbm, nbr_hbm, nodes_hbm,
                    self_out, nsum_out, nt_out, tn_out,
                    ts_v, idx_v, nidx_v, idxblk_v, self_v, rows_v, nsum_v,
                    nt_v, tn_v, sem_nbr, sem_self, sem_a, sem_b):
    wid = lax.axis_index("s") * NC + lax.axis_index("c")
    base = wid * PER_W
    sems = (sem_a, sem_b)

    pltpu.sync_copy(ts_hbm, ts_v)

    def _stage(sb):
        b = sb % 2
        for j in range(NPD):
            n = sb * NPD + j
            ia = nidx_v[n, pl.ds(0, 16)]
            ib = nidx_v[n, pl.ds(K - 16, 16)]
            idxblk_v[b, pl.ds(j * K, 16)] = ia
            idxblk_v[b, pl.ds(j * K + K - 16, 16)] = ib
            nt_v[n, pl.ds(0, 16)] = plsc.load_gather(ts_v, [ia])
            nt_v[n, pl.ds(K - 16, 16)] = plsc.load_gather(ts_v, [ib])
        return pltpu.async_copy(feat_hbm.at[idxblk_v.at[b]], rows_v.at[b],
                                sems[b])

    def _consume(sb):
        b = sb % 2
        pltpu.make_async_copy(feat_hbm.at[idxblk_v.at[b]], rows_v.at[b],
                              sems[b]).wait()

        @pl.loop(0, NPD)
        def _node(j):
            n = sb * NPD + j
            r0 = j * K
            for l in range(D // 16):
                acc = rows_v[b, r0, pl.ds(l * 16, 16)]
                for k in range(1, K):
                    acc = acc + rows_v[b, r0 + k, pl.ds(l * 16, 16)]
                nsum_v[n, pl.ds(l * 16, 16)] = acc

    @pl.loop(0, N_CHUNKS)
    def _chunk(c):
        cb = base + c * C
        pltpu.sync_copy(nodes_hbm.at[pl.ds(cb, C)], idx_v)
        nbr_cp = pltpu.async_copy(nbr_hbm.at[idx_v], nidx_v, sem_nbr)
        self_cp = pltpu.async_copy(feat_hbm.at[idx_v], self_v, sem_self)
        nbr_cp.wait()

        iv = idx_v[pl.ds(0, 16)]
        tn_v[pl.ds(0, 16)] = plsc.load_gather(ts_v, [iv])

        _stage(0)
        for sb in range(NSB):
            if sb + 1 < NSB:
                _stage(sb + 1)
            _consume(sb)

        self_cp.wait()
        pltpu.sync_copy(self_v, self_out.at[pl.ds(cb, C)])
        pltpu.sync_copy(nsum_v, nsum_out.at[pl.ds(cb, C)])
        pltpu.sync_copy(nt_v, nt_out.at[pl.ds(cb, C)])
        pltpu.sync_copy(tn_v, tn_out.at[pl.ds(cb, C)])


def _sc_gather(feat, ts, nbr, nodes):
    mesh = plsc.VectorSubcoreMesh(core_axis_name="c", subcore_axis_name="s",
                                  num_cores=NC, num_subcores=NS)
    f = pl.kernel(
        _sc_gather_body,
        compiler_params=pltpu.CompilerParams(needs_layout_passes=False,
                                             use_tc_tiling_on_sc=False),
        out_type=(
            jax.ShapeDtypeStruct((B, D), jnp.float32),
            jax.ShapeDtypeStruct((B, D), jnp.float32),
            jax.ShapeDtypeStruct((B, K), jnp.float32),
            jax.ShapeDtypeStruct((B,), jnp.float32),
        ),
        mesh=mesh,
        scratch_types=[
            pltpu.VMEM((N_NODES,), jnp.float32),
            pltpu.VMEM((C,), jnp.int32),
            pltpu.VMEM((C, KP), jnp.int32),
            pltpu.VMEM((2, NPD * K), jnp.int32),
            pltpu.VMEM((C, D), jnp.float32),
            pltpu.VMEM((2, NPD * K, D), jnp.float32),
            pltpu.VMEM((C, D), jnp.float32),
            pltpu.VMEM((C, K), jnp.float32),
            pltpu.VMEM((C,), jnp.float32),
            pltpu.SemaphoreType.DMA,
            pltpu.SemaphoreType.DMA,
            pltpu.SemaphoreType.DMA,
            pltpu.SemaphoreType.DMA,
        ],
    )
    return f(feat, ts, nbr, nodes)


RB = 512
GRID = B // RB

_INV2PI = 0.15915494309189535
_C2PI_HI = 6.28125
_C2PI_LO = 0.0019353071795864769
_COS_C = (0.9999999922836861, -0.49999991772810104, 0.04166652436596424,
          -0.001388797041345639, 2.477342421737267e-05,
          -2.7113373320120093e-07, 1.7369133908928802e-09)


def _fast_cos(x):
    k = jnp.rint(x * _INV2PI)
    r = (x - k * _C2PI_HI) - k * _C2PI_LO
    r2 = r * r
    acc = jnp.float32(_COS_C[6])
    for j in range(5, -1, -1):
        acc = acc * r2 + jnp.float32(_COS_C[j])
    return acc


def _tc_embed_body(selff_ref, nsum_ref, nt_ref, tn_ref, et_ref,
                   tw_ref, tb_ref, sw_ref, sb_ref, fw_ref, fb_ref, ne_ref):
    w = tw_ref[0:1, :]
    bvec = tb_ref[0:1, :]
    et = et_ref[:, :]

    nt = nt_ref[:, :]
    delta = et - nt
    csum = jnp.zeros((RB, D), jnp.float32)
    for k in range(K):
        csum = csum + _fast_cos(delta[:, k:k + 1] * w + bvec)
    agg = (nsum_ref[:, :] + csum) * (1.0 / K)
    sf = selff_ref[:, :]
    ge = jnp.maximum(
        jnp.dot(sf, sw_ref[0:D, :], preferred_element_type=jnp.float32)
        + jnp.dot(agg, sw_ref[D:2 * D, :], preferred_element_type=jnp.float32)
        + sb_ref[0:1, :], 0.0)
    te = sf + _fast_cos(tn_ref[:, :] * w + bvec)
    ne_ref[:, :] = (
        jnp.dot(te, fw_ref[0:D, :], preferred_element_type=jnp.float32)
        + jnp.dot(ge, fw_ref[D:2 * D, :], preferred_element_type=jnp.float32)
        + fb_ref[0:1, :])


def _tc_embed(selff, nsum, nt, tn, et, tw, tb, sw, sb, fw, fb):
    full = lambda shape: pl.BlockSpec(shape, lambda j: (0,) * len(shape))
    grid_spec = pl.GridSpec(
        grid=(GRID,),
        in_specs=[
            pl.BlockSpec((RB, D), lambda j: (j, 0)),
            pl.BlockSpec((RB, D), lambda j: (j, 0)),
            pl.BlockSpec((RB, K), lambda j: (j, 0)),
            pl.BlockSpec((RB, 1), lambda j: (j, 0)),
            pl.BlockSpec((RB, 1), lambda j: (j, 0)),
            full((1, D)), full((1, D)),
            full((2 * D, D)), full((1, D)),
            full((2 * D, D)), full((1, D)),
        ],
        out_specs=pl.BlockSpec((RB, D), lambda j: (j, 0)),
    )
    return pl.pallas_call(
        _tc_embed_body,
        grid_spec=grid_spec,
        out_shape=jax.ShapeDtypeStruct((B, D), jnp.float32),
    )(selff, nsum, nt, tn, et, tw, tb, sw, sb, fw, fb)


def _tc_merge_body(ne0_ref, ne1_ref, ne2_ref, m1_ref, mb1_ref, w2_ref,
                   mb2_ref, pos_ref, neg_ref):
    a = jnp.dot(ne0_ref[:, :], m1_ref[0:D, :],
                preferred_element_type=jnp.float32)
    hp = jnp.maximum(a + jnp.dot(ne1_ref[:, :], m1_ref[D:2 * D, :],
                                 preferred_element_type=jnp.float32)
                     + mb1_ref[0:1, :], 0.0)
    hn = jnp.maximum(a + jnp.dot(ne2_ref[:, :], m1_ref[D:2 * D, :],
                                 preferred_element_type=jnp.float32)
                     + mb1_ref[0:1, :], 0.0)
    w2 = w2_ref[0:1, :]
    b2 = mb2_ref[0, 0]
    sp = jnp.sum(hp * w2, axis=1) + b2
    sn = jnp.sum(hn * w2, axis=1) + b2
    pos_ref[:, :] = jax.nn.sigmoid(sp)[:, None]
    neg_ref[:, :] = jax.nn.sigmoid(sn)[:, None]


def _tc_merge(ne0, ne1, ne2, m1, mb1, w2, mb2):
    full = lambda shape: pl.BlockSpec(shape, lambda j: (0,) * len(shape))
    grid_spec = pl.GridSpec(
        grid=(GRID,),
        in_specs=[
            pl.BlockSpec((RB, D), lambda j: (j, 0)),
            pl.BlockSpec((RB, D), lambda j: (j, 0)),
            pl.BlockSpec((RB, D), lambda j: (j, 0)),
            full((2 * D, D)), full((1, D)),
            full((1, D)), full((1, 1)),
        ],
        out_specs=[
            pl.BlockSpec((RB, 1), lambda j: (j, 0)),
            pl.BlockSpec((RB, 1), lambda j: (j, 0)),
        ],
    )
    return pl.pallas_call(
        _tc_merge_body,
        grid_spec=grid_spec,
        out_shape=(
            jax.ShapeDtypeStruct((B, 1), jnp.float32),
            jax.ShapeDtypeStruct((B, 1), jnp.float32),
        ),
    )(ne0, ne1, ne2, m1, mb1, w2, mb2)


def kernel(src_nodes, dst_nodes, neg_nodes, edge_times, node_features,
           node_timestamps, neighbors, time_w, time_b, sage_w, sage_b,
           fc_w, fc_b, merge_w1, merge_b1, merge_w2, merge_b2):
    nbr_pad = jnp.pad(neighbors.astype(jnp.int32), ((0, 0), (0, KP - K)))
    et = edge_times.reshape(B, 1)
    tw = time_w.reshape(1, D)
    tb = time_b.reshape(1, D)
    sb = sage_b.reshape(1, D)
    fb = fc_b.reshape(1, D)

    gath = [_sc_gather(node_features, node_timestamps, nbr_pad,
                       nodes.astype(jnp.int32))
            for nodes in (src_nodes, dst_nodes, neg_nodes)]
    ne = [_tc_embed(selff, nsum, nt, tn.reshape(B, 1), et,
                    tw, tb, sage_w, sb, fc_w, fb)
          for selff, nsum, nt, tn in gath]

    pos, neg = _tc_merge(ne[0], ne[1], ne[2], merge_w1,
                         merge_b1.reshape(1, D), merge_w2.reshape(1, D),
                         merge_b2.reshape(1, 1))
    return pos.reshape(B), neg.reshape(B)


# speedup vs baseline: 10.3056x; 1.1177x over previous
import functools

import jax
import jax.numpy as jnp
from jax import lax
from jax.experimental import pallas as pl
from jax.experimental.pallas import tpu as pltpu
from jax.experimental.pallas import tpu_sc as plsc

N_NODES = 100000
D = 128
B = 4096
K = 20
NB3 = 3 * B
NC = 2
NS = 16
NW = NC * NS
PER_W = NB3 // NW
C = 16
N_CHUNKS = PER_W // C
KP = 32
NPD = 4
NSB = C // NPD


def _sc_gather_body(feat_hbm, ts_hbm, nbr_hbm, nodes_hbm,
                    self_out, nsum_out, nt_out, tn_out,
                    ts_v, idx_v, nidx_v, idxblk_v, self_v, rows_v, nsum_v,
                    nt_v, tn_v, sem_nbr, sem_self, sem_a, sem_b):
    wid = lax.axis_index("s") * NC + lax.axis_index("c")
    base = wid * PER_W
    sems = (sem_a, sem_b)

    pltpu.sync_copy(ts_hbm, ts_v)

    def _stage(sb):
        b = sb % 2
        for j in range(NPD):
            n = sb * NPD + j
            ia = nidx_v[n, pl.ds(0, 16)]
            ib = nidx_v[n, pl.ds(K - 16, 16)]
            idxblk_v[b, pl.ds(j * K, 16)] = ia
            idxblk_v[b, pl.ds(j * K + K - 16, 16)] = ib
            nt_v[n, pl.ds(0, 16)] = plsc.load_gather(ts_v, [ia])
            nt_v[n, pl.ds(K - 16, 16)] = plsc.load_gather(ts_v, [ib])
        return pltpu.async_copy(feat_hbm.at[idxblk_v.at[b]], rows_v.at[b],
                                sems[b])

    def _consume(sb):
        b = sb % 2
        pltpu.make_async_copy(feat_hbm.at[idxblk_v.at[b]], rows_v.at[b],
                              sems[b]).wait()

        @pl.loop(0, NPD)
        def _node(j):
            n = sb * NPD + j
            r0 = j * K
            for l in range(D // 16):
                acc = rows_v[b, r0, pl.ds(l * 16, 16)]
                for k in range(1, K):
                    acc = acc + rows_v[b, r0 + k, pl.ds(l * 16, 16)]
                nsum_v[n, pl.ds(l * 16, 16)] = acc

    @pl.loop(0, N_CHUNKS)
    def _chunk(c):
        cb = base + c * C
        pltpu.sync_copy(nodes_hbm.at[pl.ds(cb, C)], idx_v)
        nbr_cp = pltpu.async_copy(nbr_hbm.at[idx_v], nidx_v, sem_nbr)
        self_cp = pltpu.async_copy(feat_hbm.at[idx_v], self_v, sem_self)
        nbr_cp.wait()

        iv = idx_v[pl.ds(0, 16)]
        tn_v[pl.ds(0, 16)] = plsc.load_gather(ts_v, [iv])

        _stage(0)
        for sb in range(NSB):
            if sb + 1 < NSB:
                _stage(sb + 1)
            _consume(sb)

        self_cp.wait()
        d1 = pltpu.async_copy(self_v, self_out.at[pl.ds(cb, C)], sem_self)
        d2 = pltpu.async_copy(nsum_v, nsum_out.at[pl.ds(cb, C)], sem_nbr)
        d3 = pltpu.async_copy(nt_v, nt_out.at[pl.ds(cb, C)], sem_a)
        d4 = pltpu.async_copy(tn_v, tn_out.at[pl.ds(cb, C)], sem_b)
        d1.wait()
        d2.wait()
        d3.wait()
        d4.wait()


def _sc_gather(feat, ts, nbr, nodes):
    mesh = plsc.VectorSubcoreMesh(core_axis_name="c", subcore_axis_name="s",
                                  num_cores=NC, num_subcores=NS)
    f = pl.kernel(
        _sc_gather_body,
        compiler_params=pltpu.CompilerParams(needs_layout_passes=False,
                                             use_tc_tiling_on_sc=False),
        out_type=(
            jax.ShapeDtypeStruct((NB3, D), jnp.float32),
            jax.ShapeDtypeStruct((NB3, D), jnp.float32),
            jax.ShapeDtypeStruct((NB3, K), jnp.float32),
            jax.ShapeDtypeStruct((NB3,), jnp.float32),
        ),
        mesh=mesh,
        scratch_types=[
            pltpu.VMEM((N_NODES,), jnp.float32),
            pltpu.VMEM((C,), jnp.int32),
            pltpu.VMEM((C, KP), jnp.int32),
            pltpu.VMEM((2, NPD * K), jnp.int32),
            pltpu.VMEM((C, D), jnp.float32),
            pltpu.VMEM((2, NPD * K, D), jnp.float32),
            pltpu.VMEM((C, D), jnp.float32),
            pltpu.VMEM((C, K), jnp.float32),
            pltpu.VMEM((C,), jnp.float32),
            pltpu.SemaphoreType.DMA,
            pltpu.SemaphoreType.DMA,
            pltpu.SemaphoreType.DMA,
            pltpu.SemaphoreType.DMA,
        ],
    )
    return f(feat, ts, nbr, nodes)


RB = 512
GRID = B // RB

_INV2PI = 0.15915494309189535
_C2PI_HI = 6.28125
_C2PI_LO = 0.0019353071795864769
_COS_C = (0.9999999922836861, -0.49999991772810104, 0.04166652436596424,
          -0.001388797041345639, 2.477342421737267e-05,
          -2.7113373320120093e-07, 1.7369133908928802e-09)


def _fast_cos(x):
    k = jnp.rint(x * _INV2PI)
    r = (x - k * _C2PI_HI) - k * _C2PI_LO
    r2 = r * r
    acc = jnp.float32(_COS_C[6])
    for j in range(5, -1, -1):
        acc = acc * r2 + jnp.float32(_COS_C[j])
    return acc


def _tc_body(selff_ref, nsum_ref, nt_ref, tn_ref, et_ref,
             tw_ref, tb_ref, sw_ref, sb_ref, fw_ref, fb_ref,
             m1_ref, mb1_ref, w2_ref, mb2_ref,
             pos_ref, neg_ref):
    w = tw_ref[0:1, :]
    bvec = tb_ref[0:1, :]
    et = et_ref[:, :]

    ne = []
    for s in range(3):
        nt = nt_ref[s]
        delta = et - nt
        csum = jnp.zeros((RB, D), jnp.float32)
        for k in range(K):
            csum = csum + _fast_cos(delta[:, k:k + 1] * w + bvec)
        agg = (nsum_ref[s] + csum) * (1.0 / K)
        sf = selff_ref[s]
        ge = jnp.maximum(
            jnp.dot(sf, sw_ref[0:D, :], preferred_element_type=jnp.float32)
            + jnp.dot(agg, sw_ref[D:2 * D, :], preferred_element_type=jnp.float32)
            + sb_ref[0:1, :], 0.0)
        te = sf + _fast_cos(tn_ref[s] * w + bvec)
        ne.append(
            jnp.dot(te, fw_ref[0:D, :], preferred_element_type=jnp.float32)
            + jnp.dot(ge, fw_ref[D:2 * D, :], preferred_element_type=jnp.float32)
            + fb_ref[0:1, :])

    a = jnp.dot(ne[0], m1_ref[0:D, :], preferred_element_type=jnp.float32)
    hp = jnp.maximum(a + jnp.dot(ne[1], m1_ref[D:2 * D, :],
                                 preferred_element_type=jnp.float32)
                     + mb1_ref[0:1, :], 0.0)
    hn = jnp.maximum(a + jnp.dot(ne[2], m1_ref[D:2 * D, :],
                                 preferred_element_type=jnp.float32)
                     + mb1_ref[0:1, :], 0.0)
    w2 = w2_ref[0:1, :]
    b2 = mb2_ref[0, 0]
    sp = jnp.sum(hp * w2, axis=1) + b2
    sn = jnp.sum(hn * w2, axis=1) + b2
    pos_ref[:, :] = jax.nn.sigmoid(sp)[:, None]
    neg_ref[:, :] = jax.nn.sigmoid(sn)[:, None]


def _tc_dense(selff, nsum, nt, tn, et, tw, tb, sw, sb, fw, fb, m1, mb1, w2, mb2):
    full = lambda shape: pl.BlockSpec(shape, lambda j: (0,) * len(shape))
    grid_spec = pl.GridSpec(
        grid=(GRID,),
        in_specs=[
            pl.BlockSpec((3, RB, D), lambda j: (0, j, 0)),
            pl.BlockSpec((3, RB, D), lambda j: (0, j, 0)),
            pl.BlockSpec((3, RB, K), lambda j: (0, j, 0)),
            pl.BlockSpec((3, RB, 1), lambda j: (0, j, 0)),
            pl.BlockSpec((RB, 1), lambda j: (j, 0)),
            full((1, D)), full((1, D)),
            full((2 * D, D)), full((1, D)),
            full((2 * D, D)), full((1, D)),
            full((2 * D, D)), full((1, D)),
            full((1, D)), full((1, 1)),
        ],
        out_specs=[
            pl.BlockSpec((RB, 1), lambda j: (j, 0)),
            pl.BlockSpec((RB, 1), lambda j: (j, 0)),
        ],
    )
    return pl.pallas_call(
        _tc_body,
        grid_spec=grid_spec,
        out_shape=(
            jax.ShapeDtypeStruct((B, 1), jnp.float32),
            jax.ShapeDtypeStruct((B, 1), jnp.float32),
        ),
    )(selff, nsum, nt, tn, et, tw, tb, sw, sb, fw, fb, m1, mb1, w2, mb2)


def kernel(src_nodes, dst_nodes, neg_nodes, edge_times, node_features,
           node_timestamps, neighbors, time_w, time_b, sage_w, sage_b,
           fc_w, fc_b, merge_w1, merge_b1, merge_w2, merge_b2):
    nodes = jnp.concatenate([src_nodes, dst_nodes, neg_nodes]).astype(jnp.int32)
    nbr_pad = jnp.pad(neighbors.astype(jnp.int32), ((0, 0), (0, KP - K)))
    selff, nsum, nt, tn = _sc_gather(node_features, node_timestamps,
                                     nbr_pad, nodes)
    pos, neg = _tc_dense(
        selff.reshape(3, B, D), nsum.reshape(3, B, D), nt.reshape(3, B, K),
        tn.reshape(3, B, 1), edge_times.reshape(B, 1),
        time_w.reshape(1, D), time_b.reshape(1, D),
        sage_w, sage_b.reshape(1, D), fc_w, fc_b.reshape(1, D),
        merge_w1, merge_b1.reshape(1, D), merge_w2.reshape(1, D),
        merge_b2.reshape(1, 1))
    return pos.reshape(B), neg.reshape(B)


# speedup vs baseline: 12.9400x; 1.2556x over previous
import functools

import jax
import jax.numpy as jnp
from jax import lax
from jax.experimental import pallas as pl
from jax.experimental.pallas import tpu as pltpu
from jax.experimental.pallas import tpu_sc as plsc

N_NODES = 100000
D = 128
B = 4096
K = 20
NB3 = 3 * B
NC = 2
NS = 16
NW = NC * NS
PER_W = NB3 // NW
C = 16
N_CHUNKS = PER_W // C
KP = 32
NPD = 4
NSB = C // NPD


def _sc_gather_body(feat_hbm, ts_hbm, nbr_hbm, nodes_hbm,
                    self_out, nsum_out, nt_out, tn_out,
                    ts_v, idx_v, nidx_v, idxblk_v, self_v, rows_v, nsum_v,
                    nt_v, tn_v, sem_nbr_a, sem_nbr_b, sem_self_a, sem_self_b,
                    sem_a, sem_b, sem_o1, sem_o2, sem_o3, sem_o4):
    wid = lax.axis_index("s") * NC + lax.axis_index("c")
    base = wid * PER_W
    sems = (sem_a, sem_b)
    sems_nbr = (sem_nbr_a, sem_nbr_b)
    sems_self = (sem_self_a, sem_self_b)

    def _issue_chunk(cb, b):
        pltpu.sync_copy(nodes_hbm.at[pl.ds(cb, C)], idx_v.at[b])
        pltpu.async_copy(nbr_hbm.at[idx_v.at[b]], nidx_v.at[b], sems_nbr[b])
        pltpu.async_copy(feat_hbm.at[idx_v.at[b]], self_v.at[b],
                         sems_self[b])

    def _stage(sb, b):
        p = sb % 2
        for j in range(NPD):
            n = sb * NPD + j
            ia = nidx_v[b, n, pl.ds(0, 16)]
            ib = nidx_v[b, n, pl.ds(K - 16, 16)]
            idxblk_v[p, pl.ds(j * K, 16)] = ia
            idxblk_v[p, pl.ds(j * K + K - 16, 16)] = ib
            nt_v[n, pl.ds(0, 16)] = plsc.load_gather(ts_v, [ia])
            nt_v[n, pl.ds(K - 16, 16)] = plsc.load_gather(ts_v, [ib])
        pltpu.async_copy(feat_hbm.at[idxblk_v.at[p]], rows_v.at[p], sems[p])

    def _consume(sb):
        p = sb % 2
        pltpu.make_async_copy(feat_hbm.at[idxblk_v.at[p]], rows_v.at[p],
                              sems[p]).wait()

        @pl.loop(0, NPD)
        def _node(j):
            n = sb * NPD + j
            r0 = j * K
            acc0 = tuple(rows_v[p, r0, pl.ds(l * 16, 16)]
                         for l in range(D // 16))

            @pl.loop(1, K, init_carry=acc0)
            def acc_fin(k, acc):
                return tuple(acc[l] + rows_v[p, r0 + k, pl.ds(l * 16, 16)]
                             for l in range(D // 16))

            for l in range(D // 16):
                nsum_v[n, pl.ds(l * 16, 16)] = acc_fin[l]

    def _process(cb, b):
        pltpu.make_async_copy(nbr_hbm.at[idx_v.at[b]], nidx_v.at[b],
                              sems_nbr[b]).wait()
        iv = idx_v[b, pl.ds(0, 16)]
        tn_v[pl.ds(0, 16)] = plsc.load_gather(ts_v, [iv])

        _stage(0, b)
        for sb in range(NSB):
            if sb + 1 < NSB:
                _stage(sb + 1, b)
            _consume(sb)

        pltpu.make_async_copy(feat_hbm.at[idx_v.at[b]], self_v.at[b],
                              sems_self[b]).wait()
        d1 = pltpu.async_copy(self_v.at[b], self_out.at[pl.ds(cb, C)], sem_o1)
        d2 = pltpu.async_copy(nsum_v, nsum_out.at[pl.ds(cb, C)], sem_o2)
        d3 = pltpu.async_copy(nt_v, nt_out.at[pl.ds(cb, C)], sem_o3)
        d4 = pltpu.async_copy(tn_v, tn_out.at[pl.ds(cb, C)], sem_o4)
        d1.wait()
        d2.wait()
        d3.wait()
        d4.wait()

    _issue_chunk(base, 0)
    pltpu.sync_copy(ts_hbm, ts_v)

    @pl.loop(0, N_CHUNKS - 2, step=2)
    def _chunk(c):
        for b in range(2):
            cb = base + (c + b) * C
            _issue_chunk(cb + C, 1 - b)
            _process(cb, b)

    _issue_chunk(base + (N_CHUNKS - 1) * C, 1)
    _process(base + (N_CHUNKS - 2) * C, 0)
    _process(base + (N_CHUNKS - 1) * C, 1)


def _sc_gather(feat, ts, nbr, nodes):
    mesh = plsc.VectorSubcoreMesh(core_axis_name="c", subcore_axis_name="s",
                                  num_cores=NC, num_subcores=NS)
    f = pl.kernel(
        _sc_gather_body,
        compiler_params=pltpu.CompilerParams(needs_layout_passes=False,
                                             use_tc_tiling_on_sc=False),
        out_type=(
            jax.ShapeDtypeStruct((NB3, D), jnp.float32),
            jax.ShapeDtypeStruct((NB3, D), jnp.float32),
            jax.ShapeDtypeStruct((NB3, K), jnp.float32),
            jax.ShapeDtypeStruct((NB3,), jnp.float32),
        ),
        mesh=mesh,
        scratch_types=[
            pltpu.VMEM((N_NODES,), jnp.float32),
            pltpu.VMEM((2, C), jnp.int32),
            pltpu.VMEM((2, C, KP), jnp.int32),
            pltpu.VMEM((2, NPD * K), jnp.int32),
            pltpu.VMEM((2, C, D), jnp.float32),
            pltpu.VMEM((2, NPD * K, D), jnp.float32),
            pltpu.VMEM((C, D), jnp.float32),
            pltpu.VMEM((C, K), jnp.float32),
            pltpu.VMEM((C,), jnp.float32),
        ] + [pltpu.SemaphoreType.DMA] * 10,
    )
    return f(feat, ts, nbr, nodes)


RB = 512
GRID = B // RB

_INV2PI = 0.15915494309189535
_C2PI_HI = 6.28125
_C2PI_LO = 0.0019353071795864769
_COS_C = (0.9999999922836861, -0.49999991772810104, 0.04166652436596424,
          -0.001388797041345639, 2.477342421737267e-05,
          -2.7113373320120093e-07, 1.7369133908928802e-09)


def _fast_cos(x):
    k = jnp.rint(x * _INV2PI)
    r = (x - k * _C2PI_HI) - k * _C2PI_LO
    r2 = r * r
    acc = jnp.float32(_COS_C[6])
    for j in range(5, -1, -1):
        acc = acc * r2 + jnp.float32(_COS_C[j])
    return acc


def _tc_body(selff_ref, nsum_ref, nt_ref, tn_ref, et_ref,
             tw_ref, tb_ref, sw_ref, sb_ref, fw_ref, fb_ref,
             m1_ref, mb1_ref, w2_ref, mb2_ref,
             pos_ref, neg_ref):
    w = tw_ref[0:1, :]
    bvec = tb_ref[0:1, :]
    et = et_ref[:, :]

    ne = []
    for s in range(3):
        nt = nt_ref[s]
        delta = et - nt
        csum = jnp.zeros((RB, D), jnp.float32)
        for k in range(K):
            csum = csum + _fast_cos(delta[:, k:k + 1] * w + bvec)
        agg = (nsum_ref[s] + csum) * (1.0 / K)
        sf = selff_ref[s]
        ge = jnp.maximum(
            jnp.dot(sf, sw_ref[0:D, :], preferred_element_type=jnp.float32)
            + jnp.dot(agg, sw_ref[D:2 * D, :], preferred_element_type=jnp.float32)
            + sb_ref[0:1, :], 0.0)
        te = sf + _fast_cos(tn_ref[s] * w + bvec)
        ne.append(
            jnp.dot(te, fw_ref[0:D, :], preferred_element_type=jnp.float32)
            + jnp.dot(ge, fw_ref[D:2 * D, :], preferred_element_type=jnp.float32)
            + fb_ref[0:1, :])

    a = jnp.dot(ne[0], m1_ref[0:D, :], preferred_element_type=jnp.float32)
    hp = jnp.maximum(a + jnp.dot(ne[1], m1_ref[D:2 * D, :],
                                 preferred_element_type=jnp.float32)
                     + mb1_ref[0:1, :], 0.0)
    hn = jnp.maximum(a + jnp.dot(ne[2], m1_ref[D:2 * D, :],
                                 preferred_element_type=jnp.float32)
                     + mb1_ref[0:1, :], 0.0)
    w2 = w2_ref[0:1, :]
    b2 = mb2_ref[0, 0]
    sp = jnp.sum(hp * w2, axis=1) + b2
    sn = jnp.sum(hn * w2, axis=1) + b2
    pos_ref[:, :] = jax.nn.sigmoid(sp)[:, None]
    neg_ref[:, :] = jax.nn.sigmoid(sn)[:, None]


def _tc_dense(selff, nsum, nt, tn, et, tw, tb, sw, sb, fw, fb, m1, mb1, w2, mb2):
    full = lambda shape: pl.BlockSpec(shape, lambda j: (0,) * len(shape))
    grid_spec = pl.GridSpec(
        grid=(GRID,),
        in_specs=[
            pl.BlockSpec((3, RB, D), lambda j: (0, j, 0)),
            pl.BlockSpec((3, RB, D), lambda j: (0, j, 0)),
            pl.BlockSpec((3, RB, K), lambda j: (0, j, 0)),
            pl.BlockSpec((3, RB, 1), lambda j: (0, j, 0)),
            pl.BlockSpec((RB, 1), lambda j: (j, 0)),
            full((1, D)), full((1, D)),
            full((2 * D, D)), full((1, D)),
            full((2 * D, D)), full((1, D)),
            full((2 * D, D)), full((1, D)),
            full((1, D)), full((1, 1)),
        ],
        out_specs=[
            pl.BlockSpec((RB, 1), lambda j: (j, 0)),
            pl.BlockSpec((RB, 1), lambda j: (j, 0)),
        ],
    )
    return pl.pallas_call(
        _tc_body,
        grid_spec=grid_spec,
        out_shape=(
            jax.ShapeDtypeStruct((B, 1), jnp.float32),
            jax.ShapeDtypeStruct((B, 1), jnp.float32),
        ),
    )(selff, nsum, nt, tn, et, tw, tb, sw, sb, fw, fb, m1, mb1, w2, mb2)


def kernel(src_nodes, dst_nodes, neg_nodes, edge_times, node_features,
           node_timestamps, neighbors, time_w, time_b, sage_w, sage_b,
           fc_w, fc_b, merge_w1, merge_b1, merge_w2, merge_b2):
    nodes = jnp.concatenate([src_nodes, dst_nodes, neg_nodes]).astype(jnp.int32)
    nbr_pad = jnp.pad(neighbors.astype(jnp.int32), ((0, 0), (0, KP - K)))
    selff, nsum, nt, tn = _sc_gather(node_features, node_timestamps,
                                     nbr_pad, nodes)
    pos, neg = _tc_dense(
        selff.reshape(3, B, D), nsum.reshape(3, B, D), nt.reshape(3, B, K),
        tn.reshape(3, B, 1), edge_times.reshape(B, 1),
        time_w.reshape(1, D), time_b.reshape(1, D),
        sage_w, sage_b.reshape(1, D), fc_w, fc_b.reshape(1, D),
        merge_w1, merge_b1.reshape(1, D), merge_w2.reshape(1, D),
        merge_b2.reshape(1, 1))
    return pos.reshape(B), neg.reshape(B)
